# Initial kernel scaffold; baseline (speedup 1.0000x reference)
import jax
import jax.numpy as jnp
from jax.experimental import pallas as pl


def kernel(x, W):
    raise NotImplementedError("write your pallas kernel here")


# speedup vs baseline: 1.6431x; 1.6431x over previous
import functools
import math

import jax
import jax.numpy as jnp
from jax import lax
from jax.experimental import pallas as pl
from jax.experimental.pallas import tpu as pltpu
from jax.experimental.pallas import tpu_sc as plsc

_D = 128
_SCALE = math.sqrt(float(_D))
_NC, _NS = 2, 16
_NW = _NC * _NS
_CH = 128
_LANES = 16


@functools.lru_cache(maxsize=None)
def _build(B):
    b_per_w = B // _NW
    n_chunks = b_per_w // _CH
    assert n_chunks % 2 == 0

    mesh = plsc.VectorSubcoreMesh(
        core_axis_name="c", subcore_axis_name="s",
        num_cores=_NC, num_subcores=_NS)

    def body(w_hbm, x_hbm, out_hbm, idx_v, rows_v, sem0, sem1):
        sems = (sem0, sem1)
        wid = lax.axis_index("s") * _NC + lax.axis_index("c")
        base = wid * b_per_w
        pltpu.sync_copy(x_hbm.at[pl.ds(base, b_per_w)], idx_v)

        def fire(c, buf):
            pltpu.async_copy(w_hbm.at[idx_v.at[pl.ds(c * _CH, _CH)]],
                             rows_v.at[buf], sems[buf])

        def wait(c, buf):
            pltpu.make_async_copy(w_hbm.at[idx_v.at[pl.ds(c * _CH, _CH)]],
                                  rows_v.at[buf], sems[buf]).wait()

        fire(0, 0)

        def chunk(c, buf):
            @pl.when(c + 1 < n_chunks)
            def _():
                fire(c + 1, 1 - buf)
            wait(c, buf)

            def scale_row(r, carry):
                for s in range(_D // _LANES):
                    sl = pl.ds(s * _LANES, _LANES)
                    rows_v[buf, r, sl] = rows_v[buf, r, sl] * _SCALE
                return carry
            lax.fori_loop(0, _CH, scale_row, 0)
            pltpu.sync_copy(rows_v.at[buf],
                            out_hbm.at[pl.ds(base + c * _CH, _CH)])

        def pair(p, carry):
            chunk(2 * p, 0)
            chunk(2 * p + 1, 1)
            return carry
        lax.fori_loop(0, n_chunks // 2, pair, 0)

    return pl.kernel(
        body,
        out_type=jax.ShapeDtypeStruct((B, _D), jnp.float32),
        mesh=mesh,
        scratch_types=[
            pltpu.VMEM((b_per_w,), jnp.int32),
            pltpu.VMEM((2, _CH, _D), jnp.float32),
            pltpu.SemaphoreType.DMA,
            pltpu.SemaphoreType.DMA,
        ],
    )


def kernel(x, W):
    B = x.shape[0] * x.shape[1]
    out = _build(B)(W, x.reshape(B))
    return out.reshape(x.shape[0], x.shape[1], _D)
